# baseline (speedup 1.0000x reference)
import functools

import jax
import jax.numpy as jnp
from jax import lax
from jax.experimental import pallas as pl
from jax.experimental.pallas import tpu as pltpu
from jax.experimental.pallas import tpu_sc as plsc

_K = 512
_B = 64
_N = 8192
_L = 16
_NW = 32
_ROWS_PER_W = _B // _NW
_CHUNKS = _N // _L
_UNROLL = 8

_NB = 512
_LOWM = (1 << 22) - 1
_REG = _N // _L

_GATHER_DNUMS = lax.GatherDimensionNumbers(
    offset_dims=(), collapsed_slice_dims=(0,), start_index_map=(0,))


def _rot(v, idx):
    return lax.gather(v, idx[:, None], dimension_numbers=_GATHER_DNUMS,
                      slice_sizes=(1,),
                      mode=lax.GatherScatterMode.PROMISE_IN_BOUNDS)


def _lane_sum(v):
    iota = lax.iota(jnp.int32, _L)
    for shift in (8, 4, 2, 1):
        v = v + _rot(v, (iota + shift) & (_L - 1))
    return v


def _lane_max(v):
    iota = lax.iota(jnp.int32, _L)
    for shift in (8, 4, 2, 1):
        v = jnp.maximum(v, _rot(v, (iota + shift) & (_L - 1)))
    return v


def _body(x_hbm, out_hbm, row_v, bits_v, hist_v, ss_v, regs_v, out_v):
    wid = lax.axis_index("s") * 2 + lax.axis_index("c")
    iota = lax.iota(jnp.int32, _L)
    ones = jnp.ones((_L,), jnp.int32)
    zeros = jnp.zeros((_L,), jnp.int32)
    kvec = jnp.full((_L,), _K, jnp.int32)

    def to_scalar(vec):
        return vec[0]

    for r in range(_ROWS_PER_W):
        row = wid * _ROWS_PER_W + r
        pltpu.sync_copy(x_hbm.at[row], row_v)

        def clear(i, c):
            for u in range(_UNROLL):
                off = (i * _UNROLL + u) * _L
                hist_v[pl.ds(off, _L)] = zeros
            return c

        lax.fori_loop(0, (_NB + 1) // _UNROLL + 1, clear, jnp.int32(0))

        def prep(i, c):
            for u in range(_UNROLL):
                off = (i * _UNROLL + u) * _L
                b = lax.bitcast_convert_type(jnp.abs(row_v[pl.ds(off, _L)]),
                                             jnp.int32)
                bits_v[pl.ds(off, _L)] = b
                idx = ((b >> 22) << 4) + iota
                plsc.addupdate_scatter(hist_v, [idx], ones)
            return c

        lax.fori_loop(0, _CHUNKS // _UNROLL, prep, jnp.int32(0))

        ss_v[pl.ds(_NB * _L, _L)] = zeros

        def scan(i, acc):
            base = (31 - i) * (_L * _L)
            for u in range(_L - 1, -1, -1):
                acc = acc + hist_v[pl.ds(base + u * _L, _L)]
                ss_v[pl.ds(base + u * _L, _L)] = acc
            return acc

        lax.fori_loop(0, _NB // _L, scan, zeros)

        def bstep(i, p):
            cand = p + (jnp.int32(1) << (jnp.int32(8) - i))
            tot = _lane_sum(ss_v[pl.ds(cand * _L, _L)])
            ok = to_scalar(tot) >= _K
            return jnp.where(ok, cand, p)

        p = lax.fori_loop(0, 9, bstep, jnp.int32(0))
        p_vec = jnp.broadcast_to(p, (_L,))
        c_above = _lane_sum(ss_v[pl.ds((p + 1) * _L, _L)])
        kp_vec = kvec - c_above

        def comp(i, offs):
            for u in range(_UNROLL):
                off = (i * _UNROLL + u) * _L
                b = bits_v[pl.ds(off, _L)]
                m = (b >> 22) == p_vec
                plsc.store_scatter(regs_v, [offs], b & _LOWM, mask=m)
                offs = offs + jnp.where(m, ones, zeros)
            return offs

        offs = lax.fori_loop(0, _CHUNKS // _UNROLL, comp, iota * _REG)
        cnts = offs - iota * _REG
        nch = (to_scalar(_lane_max(cnts)) + (_L - 1)) // _L

        def tclear(j, c):
            pos = j * _L + iota
            for l in range(_L):
                cl = jnp.broadcast_to(cnts[l], (_L,))
                base = l * _REG + j * _L
                old = regs_v[pl.ds(base, _L)]
                regs_v[pl.ds(base, _L)] = jnp.where(pos < cl, old, zeros)
            return c

        lax.fori_loop(0, nch, tclear, jnp.int32(0))

        def bit_step(bi, t):
            cand = t | (ones << jnp.broadcast_to(jnp.int32(21) - bi, (_L,)))

            def cch(j, cnt):
                for l in range(_L):
                    v = regs_v[pl.ds(l * _REG + j * _L, _L)]
                    cnt = cnt + jnp.where(v >= cand, ones, zeros)
                return cnt

            cnt = lax.fori_loop(0, nch, cch, zeros)
            tot = _lane_sum(cnt)
            return jnp.where(tot >= kp_vec, cand, t)

        tlo = lax.fori_loop(0, 22, bit_step, zeros)
        thresh = (p_vec << 22) | tlo

        def mask_chunk(i, c):
            for u in range(_UNROLL):
                off = (i * _UNROLL + u) * _L
                v = row_v[pl.ds(off, _L)]
                keep = lax.bitcast_convert_type(jnp.abs(v), jnp.int32) >= thresh
                out_v[pl.ds(off, _L)] = jnp.where(keep, v, jnp.float32(0))
            return c

        lax.fori_loop(0, _CHUNKS // _UNROLL, mask_chunk, jnp.int32(0))
        pltpu.sync_copy(out_v, out_hbm.at[row])


@jax.jit
def kernel(x):
    mesh = plsc.VectorSubcoreMesh(core_axis_name="c", subcore_axis_name="s")
    fn = functools.partial(
        pl.kernel,
        mesh=mesh,
        compiler_params=pltpu.CompilerParams(needs_layout_passes=False),
        out_type=jax.ShapeDtypeStruct((_B, _N), jnp.float32),
        scratch_types=[
            pltpu.VMEM((_N,), jnp.float32),
            pltpu.VMEM((_N,), jnp.int32),
            pltpu.VMEM(((_NB + _UNROLL) * _L,), jnp.int32),
            pltpu.VMEM(((_NB + _UNROLL) * _L,), jnp.int32),
            pltpu.VMEM((_N,), jnp.int32),
            pltpu.VMEM((_N,), jnp.float32),
        ],
    )(_body)
    return fn(x)
